# baseline (speedup 1.0000x reference)
import functools

import jax
import jax.numpy as jnp
from jax import lax
from jax.experimental import pallas as pl
from jax.experimental.pallas import tpu as pltpu
from jax.experimental.pallas import tpu_sc as plsc

_N_WORKERS = 32
_CHUNK = 128


@functools.lru_cache(maxsize=None)
def _make(total: int, dim: int):
    per_w = total // _N_WORKERS
    n_chunks = per_w // _CHUNK
    mesh = plsc.VectorSubcoreMesh(core_axis_name="c", subcore_axis_name="s")

    @functools.partial(
        pl.kernel,
        out_type=jax.ShapeDtypeStruct((total, dim), jnp.float32),
        mesh=mesh,
        scratch_types=[
            pltpu.VMEM((_CHUNK,), jnp.int32),
            pltpu.VMEM((_CHUNK, dim), jnp.float32),
            pltpu.SemaphoreType.DMA,
        ],
        compiler_params=pltpu.CompilerParams(use_tc_tiling_on_sc=False),
    )
    def gather_kernel(idx_hbm, table_hbm, out_hbm, idx_v, rows_v, sem):
        wid = lax.axis_index("s") * 2 + lax.axis_index("c")
        wbase = wid * per_w

        def body(c, carry):
            base = wbase + c * _CHUNK
            pltpu.sync_copy(idx_hbm.at[pl.ds(base, _CHUNK)], idx_v)
            pltpu.async_copy(table_hbm.at[idx_v], rows_v, sem).wait()
            pltpu.sync_copy(rows_v, out_hbm.at[pl.ds(base, _CHUNK)])
            return carry

        lax.fori_loop(0, n_chunks, body, 0)

    return gather_kernel


def kernel(z, table):
    b, s = z.shape
    total = b * s
    dim = table.shape[1]
    zf = z.reshape(total).astype(jnp.int32)
    out = _make(total, dim)(zf, table)
    return out.reshape(b, s, dim)


# speedup vs baseline: 1.1288x; 1.1288x over previous
import functools

import jax
import jax.numpy as jnp
from jax import lax
from jax.experimental import pallas as pl
from jax.experimental.pallas import tpu as pltpu
from jax.experimental.pallas import tpu_sc as plsc

_N_WORKERS = 32
_CHUNK = 256
_NBUF = 4


@functools.lru_cache(maxsize=None)
def _make(total: int, dim: int):
    per_w = total // _N_WORKERS
    n_chunks = per_w // _CHUNK
    n_outer = n_chunks // _NBUF
    assert per_w * _N_WORKERS == total
    assert n_outer * _NBUF == n_chunks and n_chunks * _CHUNK == per_w
    mesh = plsc.VectorSubcoreMesh(core_axis_name="c", subcore_axis_name="s")

    @functools.partial(
        pl.kernel,
        out_type=jax.ShapeDtypeStruct((total, dim), jnp.float32),
        mesh=mesh,
        scratch_types=[
            pltpu.VMEM((per_w,), jnp.int32),
            [pltpu.VMEM((_CHUNK, dim), jnp.float32) for _ in range(_NBUF)],
            [pltpu.SemaphoreType.DMA for _ in range(_NBUF)],
            [pltpu.SemaphoreType.DMA for _ in range(_NBUF)],
        ],
        compiler_params=pltpu.CompilerParams(use_tc_tiling_on_sc=False),
    )
    def gather_kernel(idx_hbm, table_hbm, out_hbm, idx_all, rows, gsem, wsem):
        wid = lax.axis_index("s") * 2 + lax.axis_index("c")
        wbase = wid * per_w

        pltpu.sync_copy(idx_hbm.at[pl.ds(wbase, per_w)], idx_all)

        def start_gather(b, c):
            pltpu.async_copy(
                table_hbm.at[idx_all.at[pl.ds(c * _CHUNK, _CHUNK)]],
                rows[b],
                gsem[b],
            )

        def finish_chunk(b, c):
            pltpu.make_async_copy(table_hbm.at[idx_all.at[pl.ds(0, _CHUNK)]],
                                  rows[b], gsem[b]).wait()
            out_slice = out_hbm.at[pl.ds(wbase + c * _CHUNK, _CHUNK)]
            wcopy = pltpu.async_copy(rows[b], out_slice, wsem[b])
            wcopy.wait()

        for b in range(_NBUF):
            start_gather(b, b)

        def body(i, carry):
            for b in range(_NBUF):
                c = i * _NBUF + b
                finish_chunk(b, c)
                start_gather(b, c + _NBUF)
            return carry

        if n_outer > 1:
            lax.fori_loop(0, n_outer - 1, body, 0)

        for b in range(_NBUF):
            finish_chunk(b, (n_outer - 1) * _NBUF + b)

    return gather_kernel


def kernel(z, table):
    b, s = z.shape
    total = b * s
    dim = table.shape[1]
    zf = z.reshape(total).astype(jnp.int32)
    out = _make(total, dim)(zf, table)
    return out.reshape(b, s, dim)


# speedup vs baseline: 1.1798x; 1.0451x over previous
import functools

import jax
import jax.numpy as jnp
from jax import lax
from jax.experimental import pallas as pl
from jax.experimental.pallas import tpu as pltpu
from jax.experimental.pallas import tpu_sc as plsc

_N_WORKERS = 32
_CHUNK = 256
_NBUF = 4


@functools.lru_cache(maxsize=None)
def _make(total: int, dim: int):
    per_w = total // _N_WORKERS
    n_chunks = per_w // _CHUNK
    n_outer = n_chunks // _NBUF
    assert per_w * _N_WORKERS == total
    assert n_outer * _NBUF == n_chunks and n_chunks * _CHUNK == per_w
    mesh = plsc.VectorSubcoreMesh(core_axis_name="c", subcore_axis_name="s")

    @functools.partial(
        pl.kernel,
        out_type=jax.ShapeDtypeStruct((total, dim), jnp.float32),
        mesh=mesh,
        scratch_types=[
            pltpu.VMEM((per_w,), jnp.int32),
            [pltpu.VMEM((_CHUNK, dim), jnp.float32) for _ in range(_NBUF)],
            [pltpu.SemaphoreType.DMA for _ in range(_NBUF)],
            [pltpu.SemaphoreType.DMA for _ in range(_NBUF)],
        ],
        compiler_params=pltpu.CompilerParams(use_tc_tiling_on_sc=False),
    )
    def gather_kernel(idx_hbm, table_hbm, out_hbm, idx_all, rows, gsem, wsem):
        wid = lax.axis_index("s") * 2 + lax.axis_index("c")
        wbase = wid * per_w

        pltpu.sync_copy(idx_hbm.at[pl.ds(wbase, per_w)], idx_all)

        def start_gather(b, c):
            pltpu.async_copy(
                table_hbm.at[idx_all.at[pl.ds(c * _CHUNK, _CHUNK)]],
                rows[b],
                gsem[b],
            )

        def finish_chunk(b, c):
            pltpu.make_async_copy(table_hbm.at[idx_all.at[pl.ds(0, _CHUNK)]],
                                  rows[b], gsem[b]).wait()
            out_slice = out_hbm.at[pl.ds(wbase + c * _CHUNK, _CHUNK)]
            wcopy = pltpu.async_copy(rows[b], out_slice, wsem[b])
            wcopy.wait()

        for b in range(_NBUF):
            start_gather(b, b)

        def body(i, carry):
            for b in range(_NBUF):
                c = i * _NBUF + b
                finish_chunk(b, c)
                start_gather(b, c + _NBUF)
            return carry

        if n_outer > 1:
            lax.fori_loop(0, n_outer - 1, body, 0)

        for b in range(_NBUF):
            finish_chunk(b, (n_outer - 1) * _NBUF + b)

    return gather_kernel


def kernel(z, table):
    b, s = z.shape
    total = b * s
    dim = table.shape[1]
    zf = z.T.reshape(total).astype(jnp.int32)
    out = _make(total, dim)(zf, table)
    return out.reshape(s, b, dim).transpose(1, 0, 2)
